# baseline (speedup 1.0000x reference)
import functools

import jax
import jax.numpy as jnp
from jax import lax
from jax.experimental import pallas as pl
from jax.experimental.pallas import tpu as pltpu
from jax.experimental.pallas import tpu_sc as plsc

N_VOCAB = 1000004
EMB = 64
HIST = 50
BATCH = 4096
N_WORKERS = 32
PADW = 72
T2ROWS = N_VOCAB + 12

CV = 256
N_CHUNKS = 3906
K_ALL = 122
N_EXTRA = N_CHUNKS - N_WORKERS * K_ALL
V_MAIN = CV * N_CHUNKS
V_TAIL = N_VOCAB - V_MAIN

NB = 512
N_UNITS = HIST * (BATCH // NB)
K_UNITS = 13


def _transpose_tile(inbuf, outbuf1d, iotap, ncols, nvalid):
    @pl.loop(0, nvalid // 16)
    def _(v16):
        for e in range(EMB):
            vec = inbuf[e, pl.ds(v16 * 16, 16)]
            plsc.store_scatter(outbuf1d, [iotap + (v16 * (16 * PADW) + e)],
                               vec)


def _transpose_kernel(tt_hbm, tail_hbm, t2_hbm, in0, in1, tailin, out0, out1,
                      is0, is1, os0, os1):
    wid = lax.axis_index("s") * 2 + lax.axis_index("c")
    ins = (in0, in1)
    outs = (out0, out1)
    isems = (is0, is1)
    osems = (os0, os1)
    iotap = lax.iota(jnp.int32, 16) * PADW
    extra = wid < N_EXTRA

    def chunk_off(k):
        return (wid + N_WORKERS * k) * CV

    def start_in(k, b):
        pltpu.async_copy(
            tt_hbm.at[:, pl.ds(chunk_off(k), CV)], ins[b], isems[b])

    def wait_in(b):
        pltpu.make_async_copy(
            tt_hbm.at[:, pl.ds(0, CV)], ins[b], isems[b]).wait()

    def start_out(k, b):
        pltpu.async_copy(
            outs[b], t2_hbm.at[pl.ds(chunk_off(k) * PADW, CV * PADW)],
            osems[b])

    def wait_out(b):
        pltpu.make_async_copy(
            outs[b], t2_hbm.at[pl.ds(0, CV * PADW)], osems[b]).wait()

    start_in(0, 0)

    @pl.loop(0, K_ALL, step=2)
    def _(k):
        for b in (0, 1):
            kk = k + b

            @pl.when(jnp.logical_or(kk + 1 < K_ALL, extra))
            def _():
                start_in(kk + 1, 1 - b)

            wait_in(b)

            @pl.when(kk >= 2)
            def _():
                wait_out(b)

            _transpose_tile(ins[b], outs[b], iotap, CV, CV)
            start_out(kk, b)

    @pl.when(extra)
    def _():
        wait_in(0)
        wait_out(0)
        _transpose_tile(ins[0], outs[0], iotap, CV, CV)
        start_out(K_ALL, 0)

    wait_out(1)
    wait_out(0)

    @pl.when(wid == N_WORKERS - 1)
    def _():
        pltpu.async_copy(tail_hbm, tailin, is0)
        pltpu.make_async_copy(tail_hbm, tailin, is0).wait()
        _transpose_tile(tailin, out0, iotap, 128, 80)
        pltpu.sync_copy(out0.at[pl.ds(0, 80 * PADW)],
                        t2_hbm.at[pl.ds(V_MAIN * PADW, 80 * PADW)])


def _gather_kernel(xt3_hbm, t2_hbm, out_hbm, idx0, idx1, g0, g1, tbuf,
                   gs0, gs1, ixs0, ixs1, ws):
    wid = lax.axis_index("s") * 2 + lax.axis_index("c")
    idxs = (idx0, idx1)
    gbufs = (g0, g1)
    gsems = (gs0, gs1)
    ixsems = (ixs0, ixs1)
    iota = lax.iota(jnp.int32, 16)
    giota = [g * 16 + iota for g in range(4)]
    n_last = N_UNITS - N_WORKERS * (K_UNITS - 1)

    def unit(k):
        u = wid + N_WORKERS * k
        return u // (BATCH // NB), (u % (BATCH // NB)) * NB

    def start_idx(k, b):
        h, b0 = unit(k)
        pltpu.async_copy(xt3_hbm.at[h, b0 // NB], idxs[b], ixsems[b])

    def wait_idx(b):
        pltpu.make_async_copy(
            xt3_hbm.at[0, 0], idxs[b], ixsems[b]).wait()

    def fire_gathers(b):
        for kk in range(NB // 128):
            pltpu.async_copy(
                t2_hbm.at[idxs[b].at[kk]],
                gbufs[b].at[pl.ds(kk * 128, 128)], gsems[b])

    def wait_gathers(b):
        for _ in range(NB // 128):
            pltpu.make_async_copy(
                t2_hbm.at[idxs[b].at[0]],
                gbufs[b].at[pl.ds(0, 128)], gsems[b]).wait()

    def transpose_unit(b):
        @pl.loop(0, NB // 2)
        def _(v2):
            for i in range(2):
                v = v2 * 2 + i
                vsplat = jnp.zeros((16,), jnp.int32) + v
                for g in range(4):
                    vec = gbufs[b][v, pl.ds(g * 16, 16)]
                    plsc.store_scatter(tbuf, [giota[g], vsplat], vec)

    def start_write(k):
        h, b0 = unit(k)
        return pltpu.async_copy(
            tbuf.at[:, pl.ds(0, NB)], out_hbm.at[h, :, pl.ds(b0, NB)], ws)

    start_idx(0, 0)
    wait_idx(0)
    fire_gathers(0)
    start_idx(1, 1)

    wcp = None
    for k in range(K_UNITS):
        b = k % 2
        if k == K_UNITS - 1:
            @pl.when(wid < n_last)
            def _():
                wait_gathers(b)
                if wcp is not None:
                    wcp.wait()
                transpose_unit(b)
                start_write(k).wait()

            @pl.when(wid >= n_last)
            def _():
                if wcp is not None:
                    wcp.wait()
        else:
            wait_gathers(b)
            if k + 1 == K_UNITS - 1:
                @pl.when(wid < n_last)
                def _():
                    wait_idx(1 - b)
                    fire_gathers(1 - b)
            else:
                wait_idx(1 - b)
                fire_gathers(1 - b)
                if k + 2 <= K_UNITS - 1:
                    if k + 2 == K_UNITS - 1:
                        @pl.when(wid < n_last)
                        def _():
                            start_idx(k + 2, b)
                    else:
                        start_idx(k + 2, b)
            if wcp is not None:
                wcp.wait()
            transpose_unit(b)
            wcp = start_write(k)


def kernel(x, table):
    xt3 = jnp.swapaxes(x, 0, 1).reshape(
        HIST, BATCH // NB, NB // 128, 128)

    mesh = plsc.VectorSubcoreMesh(core_axis_name="c", subcore_axis_name="s")

    t2 = table

    out_t = functools.partial(
        pl.kernel,
        out_type=jax.ShapeDtypeStruct((HIST, EMB, BATCH), jnp.float32),
        mesh=mesh,
        scratch_types=[
            pltpu.VMEM((NB // 128, 128), jnp.int32),
            pltpu.VMEM((NB // 128, 128), jnp.int32),
            pltpu.VMEM((NB, EMB), jnp.float32),
            pltpu.VMEM((NB, EMB), jnp.float32),
            pltpu.VMEM((EMB, NB + 8), jnp.float32),
            pltpu.SemaphoreType.DMA,
            pltpu.SemaphoreType.DMA,
            pltpu.SemaphoreType.DMA,
            pltpu.SemaphoreType.DMA,
            pltpu.SemaphoreType.DMA,
        ],
        compiler_params=pltpu.CompilerParams(use_tc_tiling_on_sc=False,
                                             needs_layout_passes=False),
    )(_gather_kernel)(xt3, t2)

    return jnp.transpose(out_t, (2, 0, 1))


# speedup vs baseline: 1.2415x; 1.2415x over previous
import functools

import jax
import jax.numpy as jnp
from jax import lax
from jax.experimental import pallas as pl
from jax.experimental.pallas import tpu as pltpu
from jax.experimental.pallas import tpu_sc as plsc

N_VOCAB = 1000004
EMB = 64
HIST = 50
BATCH = 4096
N_WORKERS = 32
PADW = 72
T2ROWS = N_VOCAB + 12

CV = 256
N_CHUNKS = 3906
K_ALL = 122
N_EXTRA = N_CHUNKS - N_WORKERS * K_ALL
V_MAIN = CV * N_CHUNKS
V_TAIL = N_VOCAB - V_MAIN

NB = 512
N_UNITS = HIST * (BATCH // NB)
K_UNITS = 13


def _transpose_tile(inbuf, outbuf1d, iotap, ncols, nvalid):
    @pl.loop(0, nvalid // 16)
    def _(v16):
        for e in range(EMB):
            vec = inbuf[e, pl.ds(v16 * 16, 16)]
            plsc.store_scatter(outbuf1d, [iotap + (v16 * (16 * PADW) + e)],
                               vec)


def _transpose_kernel(tt_hbm, tail_hbm, t2_hbm, in0, in1, tailin, out0, out1,
                      is0, is1, os0, os1):
    wid = lax.axis_index("s") * 2 + lax.axis_index("c")
    ins = (in0, in1)
    outs = (out0, out1)
    isems = (is0, is1)
    osems = (os0, os1)
    iotap = lax.iota(jnp.int32, 16) * PADW
    extra = wid < N_EXTRA

    def chunk_off(k):
        return (wid + N_WORKERS * k) * CV

    def start_in(k, b):
        pltpu.async_copy(
            tt_hbm.at[:, pl.ds(chunk_off(k), CV)], ins[b], isems[b])

    def wait_in(b):
        pltpu.make_async_copy(
            tt_hbm.at[:, pl.ds(0, CV)], ins[b], isems[b]).wait()

    def start_out(k, b):
        pltpu.async_copy(
            outs[b], t2_hbm.at[pl.ds(chunk_off(k) * PADW, CV * PADW)],
            osems[b])

    def wait_out(b):
        pltpu.make_async_copy(
            outs[b], t2_hbm.at[pl.ds(0, CV * PADW)], osems[b]).wait()

    start_in(0, 0)

    @pl.loop(0, K_ALL, step=2)
    def _(k):
        for b in (0, 1):
            kk = k + b

            @pl.when(jnp.logical_or(kk + 1 < K_ALL, extra))
            def _():
                start_in(kk + 1, 1 - b)

            wait_in(b)

            @pl.when(kk >= 2)
            def _():
                wait_out(b)

            _transpose_tile(ins[b], outs[b], iotap, CV, CV)
            start_out(kk, b)

    @pl.when(extra)
    def _():
        wait_in(0)
        wait_out(0)
        _transpose_tile(ins[0], outs[0], iotap, CV, CV)
        start_out(K_ALL, 0)

    wait_out(1)
    wait_out(0)

    @pl.when(wid == N_WORKERS - 1)
    def _():
        pltpu.async_copy(tail_hbm, tailin, is0)
        pltpu.make_async_copy(tail_hbm, tailin, is0).wait()
        _transpose_tile(tailin, out0, iotap, 128, 80)
        pltpu.sync_copy(out0.at[pl.ds(0, 80 * PADW)],
                        t2_hbm.at[pl.ds(V_MAIN * PADW, 80 * PADW)])


def _gather_kernel(xt3_hbm, t2_hbm, out_hbm, idx0, idx1, g0, g1, tbuf,
                   gs0, gs1, ixs0, ixs1, ws):
    wid = lax.axis_index("s") * 2 + lax.axis_index("c")
    idxs = (idx0, idx1)
    gbufs = (g0, g1)
    gsems = (gs0, gs1)
    ixsems = (ixs0, ixs1)
    iota = lax.iota(jnp.int32, 16)
    giota = [g * 16 + iota for g in range(4)]
    n_last = N_UNITS - N_WORKERS * (K_UNITS - 1)

    def unit(k):
        u = wid + N_WORKERS * k
        return u // (BATCH // NB), (u % (BATCH // NB)) * NB

    def start_idx(k, b):
        h, b0 = unit(k)
        pltpu.async_copy(xt3_hbm.at[h, b0 // NB], idxs[b], ixsems[b])

    def wait_idx(b):
        pltpu.make_async_copy(
            xt3_hbm.at[0, 0], idxs[b], ixsems[b]).wait()

    def fire_gathers(b):
        for kk in range(NB // 128):
            pltpu.async_copy(
                t2_hbm.at[idxs[b].at[kk]],
                gbufs[b].at[pl.ds(kk * 128, 128)], gsems[b])

    def wait_gathers(b):
        for _ in range(NB // 128):
            pltpu.make_async_copy(
                t2_hbm.at[idxs[b].at[0]],
                gbufs[b].at[pl.ds(0, 128)], gsems[b]).wait()

    def transpose_unit(b):
        @pl.loop(0, NB // 2)
        def _(v2):
            for i in range(2):
                v = v2 * 2 + i
                vsplat = jnp.zeros((16,), jnp.int32) + v
                for g in range(4):
                    vec = gbufs[b][v, pl.ds(g * 16, 16)]
                    plsc.store_scatter(tbuf, [giota[g], vsplat], vec)

    def start_write(k):
        h, b0 = unit(k)
        return pltpu.async_copy(
            tbuf.at[:, pl.ds(0, NB)], out_hbm.at[h, :, pl.ds(b0, NB)], ws)

    start_idx(0, 0)
    wait_idx(0)
    fire_gathers(0)
    start_idx(1, 1)

    wcp = None
    for k in range(K_UNITS):
        b = k % 2
        if k == K_UNITS - 1:
            @pl.when(wid < n_last)
            def _():
                wait_gathers(b)
                if wcp is not None:
                    wcp.wait()
                transpose_unit(b)
                start_write(k).wait()

            @pl.when(wid >= n_last)
            def _():
                if wcp is not None:
                    wcp.wait()
        else:
            wait_gathers(b)
            if k + 1 == K_UNITS - 1:
                @pl.when(wid < n_last)
                def _():
                    wait_idx(1 - b)
                    fire_gathers(1 - b)
            else:
                wait_idx(1 - b)
                fire_gathers(1 - b)
                if k + 2 <= K_UNITS - 1:
                    if k + 2 == K_UNITS - 1:
                        @pl.when(wid < n_last)
                        def _():
                            start_idx(k + 2, b)
                    else:
                        start_idx(k + 2, b)
            if wcp is not None:
                wcp.wait()
            transpose_unit(b)
            wcp = start_write(k)


def kernel(x, table):
    tt = jnp.swapaxes(table, 0, 1)
    tail = jnp.swapaxes(
        jnp.pad(lax.slice(table, (V_MAIN, 0), (N_VOCAB, EMB)),
                ((0, 128 - V_TAIL), (0, 0))), 0, 1)
    xt3 = jnp.swapaxes(x, 0, 1).reshape(
        HIST, BATCH // NB, NB // 128, 128)

    mesh = plsc.VectorSubcoreMesh(core_axis_name="c", subcore_axis_name="s")

    t2flat = functools.partial(
        pl.kernel,
        out_type=jax.ShapeDtypeStruct((T2ROWS * PADW,), jnp.float32),
        mesh=mesh,
        scratch_types=[
            pltpu.VMEM((EMB, CV), jnp.float32),
            pltpu.VMEM((EMB, CV), jnp.float32),
            pltpu.VMEM((EMB, 128), jnp.float32),
            pltpu.VMEM((CV * PADW,), jnp.float32),
            pltpu.VMEM((CV * PADW,), jnp.float32),
            pltpu.SemaphoreType.DMA,
            pltpu.SemaphoreType.DMA,
            pltpu.SemaphoreType.DMA,
            pltpu.SemaphoreType.DMA,
        ],
        compiler_params=pltpu.CompilerParams(use_tc_tiling_on_sc=True,
                                             needs_layout_passes=False),
    )(_transpose_kernel)(tt, tail)
    t2 = t2flat.reshape(T2ROWS, PADW)

    out_t = functools.partial(
        pl.kernel,
        out_type=jax.ShapeDtypeStruct((HIST, EMB, BATCH), jnp.float32),
        mesh=mesh,
        scratch_types=[
            pltpu.VMEM((NB // 128, 128), jnp.int32),
            pltpu.VMEM((NB // 128, 128), jnp.int32),
            pltpu.VMEM((NB, PADW), jnp.float32),
            pltpu.VMEM((NB, PADW), jnp.float32),
            pltpu.VMEM((EMB, NB + 8), jnp.float32),
            pltpu.SemaphoreType.DMA,
            pltpu.SemaphoreType.DMA,
            pltpu.SemaphoreType.DMA,
            pltpu.SemaphoreType.DMA,
            pltpu.SemaphoreType.DMA,
        ],
        compiler_params=pltpu.CompilerParams(use_tc_tiling_on_sc=False,
                                             needs_layout_passes=False),
    )(_gather_kernel)(xt3, t2)

    return jnp.transpose(out_t, (2, 0, 1))
